# baseline (speedup 1.0000x reference)
import jax
import jax.numpy as jnp
from jax import lax
from jax.experimental import pallas as pl
from jax.experimental.pallas import tpu as pltpu
from jax.experimental.pallas import tpu_sc as plsc

B = 16384
EMB = 32
PACK = 4
ROW = PACK * EMB
NPHYS = 262144
_NC = 2
_NS = 16
_NW = _NC * _NS
_BPW = B // _NW
_CHUNK = 256
_NTASK = 2 * (_BPW // _CHUNK)

_BLK = 2048
_TQ = 16384

def _sc_gather_body(uq_h, iq_h, ut_h, it_h,
                    u_o, i_o,
                    uq_v, iq_v, bufs, sems):
    wid = lax.axis_index("s") * _NC + lax.axis_index("c")
    base = wid * _BPW
    pltpu.sync_copy(uq_h.at[pl.ds(base, _BPW)], uq_v)
    pltpu.sync_copy(iq_h.at[pl.ds(base, _BPW)], iq_v)
    tables = (ut_h, it_h)
    idxs = (uq_v, iq_v)
    outs = (u_o, i_o)

    def start(k):
        t, c = k // 2, k % 2
        return pltpu.async_copy(
            tables[t].at[idxs[t].at[pl.ds(c * _CHUNK, _CHUNK)]],
            bufs[k % 2], sems[k % 2])

    cp = start(0)
    for k in range(_NTASK):
        nxt = start(k + 1) if k + 1 < _NTASK else None
        cp.wait()
        t, c = k // 2, k % 2
        pltpu.sync_copy(bufs[k % 2],
                        outs[t].at[pl.ds(base + c * _CHUNK, _CHUNK)])
        cp = nxt


def _sc_gather(uq, iq, ut, it):
    mesh = plsc.VectorSubcoreMesh(core_axis_name="c", subcore_axis_name="s")
    row = jax.ShapeDtypeStruct((B, ROW), jnp.int32)
    k = pl.kernel(
        _sc_gather_body,
        out_type=(row, row),
        mesh=mesh,
        scratch_types=[
            pltpu.VMEM((_BPW,), jnp.int32),
            pltpu.VMEM((_BPW,), jnp.int32),
            [pltpu.VMEM((_CHUNK, ROW), jnp.int32) for _ in range(2)],
            [pltpu.SemaphoreType.DMA for _ in range(2)],
        ],
        compiler_params=pltpu.CompilerParams(use_tc_tiling_on_sc=True),
    )
    return k(uq, iq, ut, it)


def _pack_body(g0, g1, g2, g3, m0, m1, m2, m3, out_r):
    x = jnp.concatenate([g0[...], g1[...], g2[...], g3[...]], axis=0)
    m = jnp.concatenate([m0[...], m1[...], m2[...], m3[...]], axis=0)
    xu = lax.bitcast_convert_type(x, jnp.uint32)
    mu = lax.bitcast_convert_type(m, jnp.uint32)
    hi = jnp.uint32(0xFFFF0000)
    s = (xu & hi) | lax.shift_right_logical(mu, jnp.uint32(16))
    out_r[...] = lax.bitcast_convert_type(s.T, jnp.int32)


def _pack_pair(gmf, mlp):
    tg = gmf.T
    tm = mlp.T
    grid = NPHYS // _TQ
    nin = 1000000 // _TQ
    spec = [pl.BlockSpec(
        (EMB, _TQ),
        (lambda g: (lambda i: (0, jnp.minimum(g * (NPHYS // _TQ) + i, nin))))(g))
        for g in range(PACK)]
    return pl.pallas_call(
        _pack_body,
        grid=(grid,),
        in_specs=spec + spec,
        out_specs=pl.BlockSpec((_TQ, ROW), lambda i: (i, 0)),
        out_shape=jax.ShapeDtypeStruct((NPHYS, ROW), jnp.int32),
    )(tg, tg, tg, tg, tm, tm, tm, tm)


def _sel(g, m):
    return jnp.where(
        m < 1, g[:, 0 * EMB:1 * EMB],
        jnp.where(m < 2, g[:, 1 * EMB:2 * EMB],
                  jnp.where(m < 3, g[:, 2 * EMB:3 * EMB],
                            g[:, 3 * EMB:4 * EMB])))


def _decode(pair):
    pu = lax.bitcast_convert_type(pair, jnp.uint32)
    hi = lax.bitcast_convert_type(pu & jnp.uint32(0xFFFF0000), jnp.float32)
    lo = lax.bitcast_convert_type(lax.shift_left(pu, jnp.uint32(16)),
                                  jnp.float32)
    return hi, lo


def _tc_dense_body(ur_r, ir_r, u_r, i_r,
                   w1_r, b1_r, g1_r, be1_r,
                   w2_r, b2_r, g2_r, be2_r,
                   w3_r, b3_r, g3_r, be3_r,
                   wo_r, bo_r, out_r):
    inv = lax.rsqrt(jnp.float32(1.0 + 1e-5))
    ur = ur_r[...].reshape(-1, 1)
    ir = ir_r[...].reshape(-1, 1)
    ug, um = _decode(_sel(u_r[...], ur))
    ig, im = _decode(_sel(i_r[...], ir))
    gmf = ug * ig
    h = (jnp.dot(jnp.concatenate([um, im], axis=1), w1_r[...],
                 preferred_element_type=jnp.float32)
         + b1_r[...])
    h = jnp.maximum(h * inv * g1_r[...] + be1_r[...], 0.0)
    h = jnp.dot(h, w2_r[...], preferred_element_type=jnp.float32) + b2_r[...]
    h = jnp.maximum(h * inv * g2_r[...] + be2_r[...], 0.0)
    h = jnp.dot(h, w3_r[...], preferred_element_type=jnp.float32) + b3_r[...]
    h = jnp.maximum(h * inv * g3_r[...] + be3_r[...], 0.0)
    wo = wo_r[...]
    logits = (jnp.sum(gmf * wo[0:1, :], axis=1)
              + jnp.sum(h * wo[1:2, :], axis=1)
              + bo_r[0])
    out_r[...] = jax.nn.sigmoid(logits)


def _tc_dense(ur, ir, u, i, W1, b1, g1, be1, W2, b2, g2, be2,
              W3, b3, g3, be3, Wo, bo):
    wo2 = Wo[:, 0].reshape(2, EMB)

    ispec = pl.BlockSpec((_BLK,), lambda j: (j,))
    bspec = pl.BlockSpec((_BLK, ROW), lambda j: (j, 0))
    wfull = lambda a: pl.BlockSpec(a.shape, lambda j: (0,) * a.ndim)
    grid = B // _BLK
    return pl.pallas_call(
        _tc_dense_body,
        grid=(grid,),
        in_specs=[ispec, ispec, bspec, bspec,
                  wfull(W1), wfull(b1), wfull(g1), wfull(be1),
                  wfull(W2), wfull(b2), wfull(g2), wfull(be2),
                  wfull(W3), wfull(b3), wfull(g3), wfull(be3),
                  wfull(wo2), wfull(bo)],
        out_specs=pl.BlockSpec((_BLK,), lambda j: (j,)),
        out_shape=jax.ShapeDtypeStruct((B,), jnp.float32),
    )(ur, ir, u, i, W1, b1, g1, be1, W2, b2, g2, be2,
      W3, b3, g3, be3, wo2, bo)


def kernel(user_idx, item_idx, user_emb_gmf, item_emb_gmf, user_emb_mlp,
           item_emb_mlp, W1, b1, g1, be1, W2, b2, g2, be2, W3, b3, g3, be3,
           Wo, bo):
    uq = lax.bitwise_and(user_idx, NPHYS - 1)
    iq = lax.bitwise_and(item_idx, NPHYS - 1)
    ur = lax.shift_right_logical(user_idx, 18)
    ir = lax.shift_right_logical(item_idx, 18)
    ut = _pack_pair(user_emb_gmf, user_emb_mlp)
    it = _pack_pair(item_emb_gmf, item_emb_mlp)
    u, i = _sc_gather(uq, iq, ut, it)
    return _tc_dense(ur, ir, u, i, W1, b1, g1, be1, W2, b2, g2, be2,
                     W3, b3, g3, be3, Wo, bo)


# speedup vs baseline: 1.0021x; 1.0021x over previous
import jax
import jax.numpy as jnp
from jax import lax
from jax.experimental import pallas as pl
from jax.experimental.pallas import tpu as pltpu
from jax.experimental.pallas import tpu_sc as plsc

B = 16384
EMB = 32
PACK = 4
ROW = PACK * EMB
NPHYS = 262144
_NC = 2
_NS = 16
_NW = _NC * _NS
_BPW = B // _NW
_CHUNK = 256
_NTASK = _BPW // _CHUNK

_BLK = 2048
_TQ = 16384

def _sc_gather_body(q_h, t_h, out_o, q_v, bufs, sems):
    wid = lax.axis_index("s") * _NC + lax.axis_index("c")
    base = wid * _BPW
    pltpu.sync_copy(q_h.at[pl.ds(base, _BPW)], q_v)

    def start(c):
        return pltpu.async_copy(
            t_h.at[q_v.at[pl.ds(c * _CHUNK, _CHUNK)]],
            bufs[c % 2], sems[c % 2])

    cp = start(0)
    for c in range(_NTASK):
        nxt = start(c + 1) if c + 1 < _NTASK else None
        cp.wait()
        pltpu.sync_copy(bufs[c % 2],
                        out_o.at[pl.ds(base + c * _CHUNK, _CHUNK)])
        cp = nxt


def _sc_gather(q, t):
    mesh = plsc.VectorSubcoreMesh(core_axis_name="c", subcore_axis_name="s")
    row = jax.ShapeDtypeStruct((B, ROW), jnp.int32)
    k = pl.kernel(
        _sc_gather_body,
        out_type=row,
        mesh=mesh,
        scratch_types=[
            pltpu.VMEM((_BPW,), jnp.int32),
            [pltpu.VMEM((_CHUNK, ROW), jnp.int32) for _ in range(2)],
            [pltpu.SemaphoreType.DMA for _ in range(2)],
        ],
        compiler_params=pltpu.CompilerParams(use_tc_tiling_on_sc=True),
    )
    return k(q, t)


def _pack_body(g0, g1, g2, g3, m0, m1, m2, m3, out_r):
    x = jnp.concatenate([g0[...], g1[...], g2[...], g3[...]], axis=0)
    m = jnp.concatenate([m0[...], m1[...], m2[...], m3[...]], axis=0)
    xu = lax.bitcast_convert_type(x, jnp.uint32)
    mu = lax.bitcast_convert_type(m, jnp.uint32)
    hi = jnp.uint32(0xFFFF0000)
    s = (xu & hi) | lax.shift_right_logical(mu, jnp.uint32(16))
    out_r[...] = lax.bitcast_convert_type(s.T, jnp.int32)


def _pack_pair(gmf, mlp):
    tg = gmf.T
    tm = mlp.T
    grid = NPHYS // _TQ
    nin = 1000000 // _TQ
    spec = [pl.BlockSpec(
        (EMB, _TQ),
        (lambda g: (lambda i: (0, jnp.minimum(g * (NPHYS // _TQ) + i, nin))))(g))
        for g in range(PACK)]
    return pl.pallas_call(
        _pack_body,
        grid=(grid,),
        in_specs=spec + spec,
        out_specs=pl.BlockSpec((_TQ, ROW), lambda i: (i, 0)),
        out_shape=jax.ShapeDtypeStruct((NPHYS, ROW), jnp.int32),
    )(tg, tg, tg, tg, tm, tm, tm, tm)


def _sel(g, m):
    return jnp.where(
        m < 1, g[:, 0 * EMB:1 * EMB],
        jnp.where(m < 2, g[:, 1 * EMB:2 * EMB],
                  jnp.where(m < 3, g[:, 2 * EMB:3 * EMB],
                            g[:, 3 * EMB:4 * EMB])))


def _decode(pair):
    pu = lax.bitcast_convert_type(pair, jnp.uint32)
    hi = lax.bitcast_convert_type(pu & jnp.uint32(0xFFFF0000), jnp.float32)
    lo = lax.bitcast_convert_type(lax.shift_left(pu, jnp.uint32(16)),
                                  jnp.float32)
    return hi, lo


def _tc_dense_body(ur_r, ir_r, u_r, i_r,
                   w1_r, b1_r, g1_r, be1_r,
                   w2_r, b2_r, g2_r, be2_r,
                   w3_r, b3_r, g3_r, be3_r,
                   wo_r, bo_r, out_r):
    inv = lax.rsqrt(jnp.float32(1.0 + 1e-5))
    ur = ur_r[...].reshape(-1, 1)
    ir = ir_r[...].reshape(-1, 1)
    ug, um = _decode(_sel(u_r[...], ur))
    ig, im = _decode(_sel(i_r[...], ir))
    gmf = ug * ig
    h = (jnp.dot(jnp.concatenate([um, im], axis=1), w1_r[...],
                 preferred_element_type=jnp.float32)
         + b1_r[...])
    h = jnp.maximum(h * inv * g1_r[...] + be1_r[...], 0.0)
    h = jnp.dot(h, w2_r[...], preferred_element_type=jnp.float32) + b2_r[...]
    h = jnp.maximum(h * inv * g2_r[...] + be2_r[...], 0.0)
    h = jnp.dot(h, w3_r[...], preferred_element_type=jnp.float32) + b3_r[...]
    h = jnp.maximum(h * inv * g3_r[...] + be3_r[...], 0.0)
    wo = wo_r[...]
    logits = (jnp.sum(gmf * wo[0:1, :], axis=1)
              + jnp.sum(h * wo[1:2, :], axis=1)
              + bo_r[0])
    out_r[...] = jax.nn.sigmoid(logits)


def _tc_dense(ur, ir, u, i, W1, b1, g1, be1, W2, b2, g2, be2,
              W3, b3, g3, be3, Wo, bo):
    wo2 = Wo[:, 0].reshape(2, EMB)

    ispec = pl.BlockSpec((_BLK,), lambda j: (j,))
    bspec = pl.BlockSpec((_BLK, ROW), lambda j: (j, 0))
    wfull = lambda a: pl.BlockSpec(a.shape, lambda j: (0,) * a.ndim)
    grid = B // _BLK
    return pl.pallas_call(
        _tc_dense_body,
        grid=(grid,),
        in_specs=[ispec, ispec, bspec, bspec,
                  wfull(W1), wfull(b1), wfull(g1), wfull(be1),
                  wfull(W2), wfull(b2), wfull(g2), wfull(be2),
                  wfull(W3), wfull(b3), wfull(g3), wfull(be3),
                  wfull(wo2), wfull(bo)],
        out_specs=pl.BlockSpec((_BLK,), lambda j: (j,)),
        out_shape=jax.ShapeDtypeStruct((B,), jnp.float32),
    )(ur, ir, u, i, W1, b1, g1, be1, W2, b2, g2, be2,
      W3, b3, g3, be3, wo2, bo)


def kernel(user_idx, item_idx, user_emb_gmf, item_emb_gmf, user_emb_mlp,
           item_emb_mlp, W1, b1, g1, be1, W2, b2, g2, be2, W3, b3, g3, be3,
           Wo, bo):
    uq = lax.bitwise_and(user_idx, NPHYS - 1)
    iq = lax.bitwise_and(item_idx, NPHYS - 1)
    ur = lax.shift_right_logical(user_idx, 18)
    ir = lax.shift_right_logical(item_idx, 18)
    ut = _pack_pair(user_emb_gmf, user_emb_mlp)
    u = _sc_gather(uq, ut)
    it = _pack_pair(item_emb_gmf, item_emb_mlp)
    i = _sc_gather(iq, it)
    return _tc_dense(ur, ir, u, i, W1, b1, g1, be1, W2, b2, g2, be2,
                     W3, b3, g3, be3, Wo, bo)
